# baseline (speedup 1.0000x reference)
import functools

import jax
import jax.numpy as jnp
from jax import lax
from jax.experimental import pallas as pl
from jax.experimental.pallas import tpu as pltpu
from jax.experimental.pallas import tpu_sc as plsc

N = 10000
D = 128
DO = 40
DOP = 128
E = 320000
EPS = 1e-5

NC = 2
NS = 16
NP = 10240
CH = 64
NCHUNK = 5120
EP = NCHUNK * CH
CPW = NCHUNK // (NC * NS)
RPT = NP // NS

BM = 256


def _sc_mesh():
    return plsc.VectorSubcoreMesh(core_axis_name="c", subcore_axis_name="s",
                                  num_cores=NC, num_subcores=NS)


@functools.cache
def _deg_kernel():
    @functools.partial(
        pl.kernel,
        out_type=jax.ShapeDtypeStruct((NC, NP, D), jnp.float32),
        mesh=_sc_mesh(),
        scratch_types=[
            pltpu.VMEM((CPW, CH), jnp.int32),
            pltpu.VMEM((CH, D), jnp.float32),
            pltpu.VMEM_SHARED((NP, D), jnp.float32),
            pltpu.SemaphoreType.DMA,
        ],
    )
    def _deg(dst_hbm, ones_hbm, zeros_hbm, out_hbm, dstv, onesv, degsp, dsem):
        c = lax.axis_index("c")
        s = lax.axis_index("s")
        pltpu.sync_copy(ones_hbm, onesv)
        pltpu.sync_copy(zeros_hbm.at[pl.ds(s * RPT, RPT)],
                        degsp.at[pl.ds(s * RPT, RPT)])
        plsc.subcore_barrier()
        base = (c * NS + s) * CPW
        pltpu.sync_copy(dst_hbm.at[pl.ds(base, CPW)], dstv)

        @pl.loop(0, CPW)
        def _(j):
            pltpu.async_copy(onesv, degsp.at[dstv.at[j]], dsem, add=True)

        @pl.loop(0, CPW)
        def _(j):
            pltpu.make_async_copy(onesv, degsp.at[dstv.at[j]], dsem).wait()

        plsc.subcore_barrier()
        pltpu.sync_copy(degsp.at[pl.ds(s * RPT, RPT)],
                        out_hbm.at[c, pl.ds(s * RPT, RPT)])

    return _deg


SPLIT0 = 90
SPLIT1 = 2 * CPW - SPLIT0
CPWMAX = max(SPLIT0, SPLIT1)


@functools.cache
def _make_scatter(dc):
    @functools.partial(
        pl.kernel,
        out_type=jax.ShapeDtypeStruct((NC, NP, dc), jnp.float32),
        mesh=_sc_mesh(),
        scratch_types=[
            pltpu.VMEM((CPWMAX, CH), jnp.int32),
            pltpu.VMEM((CPWMAX, CH), jnp.int32),
            pltpu.VMEM((2, CH, dc), jnp.float32),
            pltpu.SemaphoreType.DMA((2,)),
            pltpu.VMEM_SHARED((NP, dc), jnp.float32),
        ],
        compiler_params=pltpu.CompilerParams(use_tc_tiling_on_sc=False),
    )
    def _scatter(src_hbm, dst_hbm, z_hbm, zeros_hbm, out_hbm,
                 srcv, dstv, rows, gsems, acc):
        c = lax.axis_index("c")
        s = lax.axis_index("s")
        cpw = jnp.where(c == 0, SPLIT0, SPLIT1)
        base = jnp.where(c == 0, s * SPLIT0, NS * SPLIT0 + s * SPLIT1)
        pltpu.sync_copy(src_hbm.at[pl.ds(base, CPWMAX)], srcv)
        pltpu.sync_copy(dst_hbm.at[pl.ds(base, CPWMAX)], dstv)
        pltpu.sync_copy(zeros_hbm.at[pl.ds(s * RPT, RPT)],
                        acc.at[pl.ds(s * RPT, RPT)])
        plsc.subcore_barrier()

        def _gather(g, b):
            pltpu.async_copy(z_hbm.at[srcv.at[g]], rows.at[b], gsems.at[b])

        def _gwait(g, b):
            pltpu.make_async_copy(z_hbm.at[srcv.at[g]], rows.at[b],
                                  gsems.at[b]).wait()

        @pl.loop(0, CPWMAX + 1)
        def _(g):
            @pl.when(g < cpw)
            def _():
                _gather(g, lax.rem(g, 2))

            @pl.when((g > 0) & (g <= cpw))
            def _():
                b = lax.rem(g + 1, 2)
                _gwait(g - 1, b)
                pltpu.sync_copy(rows.at[b], acc.at[dstv.at[g - 1]], add=True)

        plsc.subcore_barrier()
        pltpu.sync_copy(acc.at[pl.ds(s * RPT, RPT)],
                        out_hbm.at[c, pl.ds(s * RPT, RPT)])

    return _scatter


def _tc_first(x_ref, w_ref, degp_ref, z_ref, dinv_ref):
    dp = degp_ref[0] + degp_ref[1]
    deg = dp[:, 0:1] + 1.0
    dinv = lax.rsqrt(jnp.maximum(deg, 1.0))
    dinvb = jnp.broadcast_to(dinv, (BM, D))
    xw = jnp.dot(x_ref[...], w_ref[...], preferred_element_type=jnp.float32)
    z_ref[...] = dinvb * xw
    dinv_ref[...] = dinvb


def _tc_mid(a_ref, z_ref, dinv_ref, b_ref, g_ref, be_ref, w_ref, zo_ref, dc):
    dinvb = dinv_ref[...]
    u = dinvb * (a_ref[0] + a_ref[1] + z_ref[...]) + b_ref[...]
    u = u * (g_ref[...] / jnp.sqrt(1.0 + EPS)) + be_ref[...]
    u = jnp.maximum(u, 0.0)
    zw = jnp.dot(u, w_ref[...], preferred_element_type=jnp.float32)
    zo_ref[...] = dinvb[:, :dc] * zw


def _tc_last(a_ref, z_ref, dinv_ref, b_ref, o_ref):
    logits = dinv_ref[...] * (a_ref[0] + a_ref[1] + z_ref[...]) + b_ref[...]
    col = lax.broadcasted_iota(jnp.int32, (BM, DOP), 1)
    valid = col < DO
    masked = jnp.where(valid, logits, -1e30)
    m = jnp.max(masked, axis=1, keepdims=True)
    sh = logits - m
    ex = jnp.where(valid, jnp.exp(sh), 0.0)
    lse = jnp.log(jnp.sum(ex, axis=1, keepdims=True))
    o_ref[...] = sh - lse


def _row_spec(dc):
    return pl.BlockSpec((BM, dc), lambda i: (i, 0))


def _full_spec(shape):
    return pl.BlockSpec(shape, lambda i: tuple(0 for _ in shape))


def _pair_spec(dc):
    return pl.BlockSpec((2, BM, dc), lambda i: (0, i, 0))


_GRID = (NP // BM,)


def kernel(x, adj_t, W1, b1, W2, b2, W3, b3, g1, be1, g2, be2):
    f32 = jnp.float32
    src = jnp.concatenate(
        [adj_t[0], jnp.zeros((EP - E,), jnp.int32)]).reshape(NCHUNK, CH)
    dst = jnp.concatenate(
        [adj_t[1], jnp.full((EP - E,), N, jnp.int32)]).reshape(NCHUNK, CH)
    x_p = jnp.zeros((NP, D), f32).at[:N].set(x)
    w3p = jnp.zeros((D, DOP), f32).at[:, :DO].set(W3)
    b3p = jnp.zeros((DOP,), f32).at[:DO].set(b3)
    ones_src = jnp.ones((CH, D), f32)
    zeros128 = jnp.zeros((NP, D), f32)
    zeros64 = jnp.zeros((NP, DOP), f32)

    scatter128 = _make_scatter(D)
    degp = _deg_kernel()(dst, ones_src, zeros128)

    z1, dinvb = pl.pallas_call(
        _tc_first,
        grid=_GRID,
        in_specs=[_row_spec(D), _full_spec((D, D)), _pair_spec(D)],
        out_specs=[_row_spec(D), _row_spec(D)],
        out_shape=[jax.ShapeDtypeStruct((NP, D), f32),
                   jax.ShapeDtypeStruct((NP, D), f32)],
    )(x_p, W1, degp)

    a1 = scatter128(src, dst, z1, zeros128)

    z2 = pl.pallas_call(
        functools.partial(_tc_mid, dc=D),
        grid=_GRID,
        in_specs=[_pair_spec(D), _row_spec(D), _row_spec(D),
                  _full_spec((D,)), _full_spec((D,)), _full_spec((D,)),
                  _full_spec((D, D))],
        out_specs=_row_spec(D),
        out_shape=jax.ShapeDtypeStruct((NP, D), f32),
    )(a1, z1, dinvb, b1, g1, be1, W2)

    a2 = scatter128(src, dst, z2, zeros128)

    z3 = pl.pallas_call(
        functools.partial(_tc_mid, dc=DOP),
        grid=_GRID,
        in_specs=[_pair_spec(D), _row_spec(D), _row_spec(D),
                  _full_spec((D,)), _full_spec((D,)), _full_spec((D,)),
                  _full_spec((D, DOP))],
        out_specs=_row_spec(DOP),
        out_shape=jax.ShapeDtypeStruct((NP, DOP), f32),
    )(a2, z2, dinvb, b2, g2, be2, w3p)

    a3 = scatter128(src, dst, z3, zeros64)

    out = pl.pallas_call(
        _tc_last,
        grid=_GRID,
        in_specs=[_pair_spec(DOP), _row_spec(DOP), _row_spec(DOP),
                  _full_spec((DOP,))],
        out_specs=_row_spec(DOP),
        out_shape=jax.ShapeDtypeStruct((NP, DOP), f32),
    )(a3, z3, dinvb, b3p)

    return out[:N, :DO]


# speedup vs baseline: 1.1356x; 1.1356x over previous
import functools

import jax
import jax.numpy as jnp
from jax import lax
from jax.experimental import pallas as pl
from jax.experimental.pallas import tpu as pltpu
from jax.experimental.pallas import tpu_sc as plsc

N = 10000
D = 128
DO = 40
DOP = 128
E = 320000
EPS = 1e-5

NC = 2
NS = 16
NP = 10240
CH = 64
NCHUNK = 5120
EP = NCHUNK * CH
CPW = NCHUNK // (NC * NS)
RPT = NP // NS

BM = 256


def _sc_mesh():
    return plsc.VectorSubcoreMesh(core_axis_name="c", subcore_axis_name="s",
                                  num_cores=NC, num_subcores=NS)


@functools.cache
def _deg_kernel():
    @functools.partial(
        pl.kernel,
        out_type=jax.ShapeDtypeStruct((NC, NP, D), jnp.float32),
        mesh=_sc_mesh(),
        scratch_types=[
            pltpu.VMEM((CPW, CH), jnp.int32),
            pltpu.VMEM((CH, D), jnp.float32),
            pltpu.VMEM_SHARED((NP, D), jnp.float32),
            pltpu.SemaphoreType.DMA,
        ],
    )
    def _deg(dst_hbm, ones_hbm, zeros_hbm, out_hbm, dstv, onesv, degsp, dsem):
        c = lax.axis_index("c")
        s = lax.axis_index("s")
        pltpu.sync_copy(ones_hbm, onesv)
        pltpu.sync_copy(zeros_hbm.at[pl.ds(s * RPT, RPT)],
                        degsp.at[pl.ds(s * RPT, RPT)])
        plsc.subcore_barrier()
        base = (c * NS + s) * CPW
        pltpu.sync_copy(dst_hbm.at[pl.ds(base, CPW)], dstv)

        @pl.loop(0, CPW)
        def _(j):
            pltpu.async_copy(onesv, degsp.at[dstv.at[j]], dsem, add=True)

        @pl.loop(0, CPW)
        def _(j):
            pltpu.make_async_copy(onesv, degsp.at[dstv.at[j]], dsem).wait()

        plsc.subcore_barrier()
        pltpu.sync_copy(degsp.at[pl.ds(s * RPT, RPT)],
                        out_hbm.at[c, pl.ds(s * RPT, RPT)])

    return _deg


SPLIT0 = 230
SPLIT1 = 2 * CPW - SPLIT0
CPWMAX = max(SPLIT0, SPLIT1)
NCH_PAD = NCHUNK + 256


@functools.cache
def _make_scatter(dc):
    @functools.partial(
        pl.kernel,
        out_type=jax.ShapeDtypeStruct((NC, NP, dc), jnp.float32),
        mesh=_sc_mesh(),
        scratch_types=[
            pltpu.VMEM((CPWMAX, CH), jnp.int32),
            pltpu.VMEM((CPWMAX, CH), jnp.int32),
            pltpu.VMEM((2, CH, dc), jnp.float32),
            pltpu.SemaphoreType.DMA((2,)),
            pltpu.VMEM_SHARED((NP, dc), jnp.float32),
        ],
        compiler_params=pltpu.CompilerParams(use_tc_tiling_on_sc=False),
    )
    def _scatter(src_hbm, dst_hbm, z_hbm, zeros_hbm, out_hbm,
                 srcv, dstv, rows, gsems, acc):
        c = lax.axis_index("c")
        s = lax.axis_index("s")
        cpw = jnp.where(c == 0, SPLIT0, SPLIT1)
        base = jnp.where(c == 0, s * SPLIT0, NS * SPLIT0 + s * SPLIT1)
        pltpu.sync_copy(src_hbm.at[pl.ds(base, CPWMAX)], srcv)
        pltpu.sync_copy(dst_hbm.at[pl.ds(base, CPWMAX)], dstv)
        pltpu.sync_copy(zeros_hbm.at[pl.ds(s * RPT, RPT)],
                        acc.at[pl.ds(s * RPT, RPT)])
        plsc.subcore_barrier()

        def _gather(g, b):
            pltpu.async_copy(z_hbm.at[srcv.at[g]], rows.at[b], gsems.at[b])

        def _gwait(g, b):
            pltpu.make_async_copy(z_hbm.at[srcv.at[g]], rows.at[b],
                                  gsems.at[b]).wait()

        @pl.loop(0, CPWMAX + 1)
        def _(g):
            @pl.when(g < cpw)
            def _():
                _gather(g, lax.rem(g, 2))

            @pl.when((g > 0) & (g <= cpw))
            def _():
                b = lax.rem(g + 1, 2)
                _gwait(g - 1, b)
                pltpu.sync_copy(rows.at[b], acc.at[dstv.at[g - 1]], add=True)

        plsc.subcore_barrier()
        pltpu.sync_copy(acc.at[pl.ds(s * RPT, RPT)],
                        out_hbm.at[c, pl.ds(s * RPT, RPT)])

    return _scatter


def _tc_first(x_ref, w_ref, degp_ref, z_ref, dinv_ref):
    dp = degp_ref[0] + degp_ref[1]
    deg = dp[:, 0:1] + 1.0
    dinv = lax.rsqrt(jnp.maximum(deg, 1.0))
    dinvb = jnp.broadcast_to(dinv, (BM, D))
    xw = jnp.dot(x_ref[...], w_ref[...], preferred_element_type=jnp.float32)
    z_ref[...] = dinvb * xw
    dinv_ref[...] = dinvb


def _tc_mid(a_ref, z_ref, dinv_ref, b_ref, g_ref, be_ref, w_ref, zo_ref, dc):
    dinvb = dinv_ref[...]
    u = dinvb * (a_ref[0] + a_ref[1] + z_ref[...]) + b_ref[...]
    u = u * (g_ref[...] / jnp.sqrt(1.0 + EPS)) + be_ref[...]
    u = jnp.maximum(u, 0.0)
    zw = jnp.dot(u, w_ref[...], preferred_element_type=jnp.float32)
    zo_ref[...] = dinvb[:, :dc] * zw


def _tc_last(a_ref, z_ref, dinv_ref, b_ref, o_ref):
    logits = dinv_ref[...] * (a_ref[0] + a_ref[1] + z_ref[...]) + b_ref[...]
    col = lax.broadcasted_iota(jnp.int32, (BM, DOP), 1)
    valid = col < DO
    masked = jnp.where(valid, logits, -1e30)
    m = jnp.max(masked, axis=1, keepdims=True)
    sh = logits - m
    ex = jnp.where(valid, jnp.exp(sh), 0.0)
    lse = jnp.log(jnp.sum(ex, axis=1, keepdims=True))
    o_ref[...] = sh - lse


def _row_spec(dc):
    return pl.BlockSpec((BM, dc), lambda i: (i, 0))


def _full_spec(shape):
    return pl.BlockSpec(shape, lambda i: tuple(0 for _ in shape))


def _pair_spec(dc):
    return pl.BlockSpec((2, BM, dc), lambda i: (0, i, 0))


_GRID = (NP // BM,)


def kernel(x, adj_t, W1, b1, W2, b2, W3, b3, g1, be1, g2, be2):
    f32 = jnp.float32
    src = jnp.concatenate(
        [adj_t[0],
         jnp.zeros((NCH_PAD * CH - E,), jnp.int32)]).reshape(NCH_PAD, CH)
    dst = jnp.concatenate(
        [adj_t[1], jnp.full((EP - E,), N, jnp.int32),
         jnp.zeros(((NCH_PAD - NCHUNK) * CH,), jnp.int32)]).reshape(NCH_PAD,
                                                                    CH)
    x_p = jnp.zeros((NP, D), f32).at[:N].set(x)
    w3p = jnp.zeros((D, DOP), f32).at[:, :DO].set(W3)
    b3p = jnp.zeros((DOP,), f32).at[:DO].set(b3)
    ones_src = jnp.ones((CH, D), f32)
    zeros128 = jnp.zeros((NP, D), f32)
    zeros64 = jnp.zeros((NP, DOP), f32)

    scatter128 = _make_scatter(D)
    degp = _deg_kernel()(dst, ones_src, zeros128)

    z1, dinvb = pl.pallas_call(
        _tc_first,
        grid=_GRID,
        in_specs=[_row_spec(D), _full_spec((D, D)), _pair_spec(D)],
        out_specs=[_row_spec(D), _row_spec(D)],
        out_shape=[jax.ShapeDtypeStruct((NP, D), f32),
                   jax.ShapeDtypeStruct((NP, D), f32)],
    )(x_p, W1, degp)

    a1 = scatter128(src, dst, z1, zeros128)

    z2 = pl.pallas_call(
        functools.partial(_tc_mid, dc=D),
        grid=_GRID,
        in_specs=[_pair_spec(D), _row_spec(D), _row_spec(D),
                  _full_spec((D,)), _full_spec((D,)), _full_spec((D,)),
                  _full_spec((D, D))],
        out_specs=_row_spec(D),
        out_shape=jax.ShapeDtypeStruct((NP, D), f32),
    )(a1, z1, dinvb, b1, g1, be1, W2)

    a2 = scatter128(src, dst, z2, zeros128)

    z3 = pl.pallas_call(
        functools.partial(_tc_mid, dc=DOP),
        grid=_GRID,
        in_specs=[_pair_spec(D), _row_spec(D), _row_spec(D),
                  _full_spec((D,)), _full_spec((D,)), _full_spec((D,)),
                  _full_spec((D, DOP))],
        out_specs=_row_spec(DOP),
        out_shape=jax.ShapeDtypeStruct((NP, DOP), f32),
    )(a2, z2, dinvb, b2, g2, be2, w3p)

    a3 = scatter128(src, dst, z3, zeros64)

    out = pl.pallas_call(
        _tc_last,
        grid=_GRID,
        in_specs=[_pair_spec(DOP), _row_spec(DOP), _row_spec(DOP),
                  _full_spec((DOP,))],
        out_specs=_row_spec(DOP),
        out_shape=jax.ShapeDtypeStruct((NP, DOP), f32),
    )(a3, z3, dinvb, b3p)

    return out[:N, :DO]


# speedup vs baseline: 1.1365x; 1.0007x over previous
import functools

import jax
import jax.numpy as jnp
from jax import lax
from jax.experimental import pallas as pl
from jax.experimental.pallas import tpu as pltpu
from jax.experimental.pallas import tpu_sc as plsc

N = 10000
D = 128
DO = 40
DOP = 128
E = 320000
EPS = 1e-5

NC = 2
NS = 16
NP = 10240
CH = 64
NCHUNK = 5120
EP = NCHUNK * CH
CPW = NCHUNK // (NC * NS)
RPT = NP // NS

BM = 256


def _sc_mesh():
    return plsc.VectorSubcoreMesh(core_axis_name="c", subcore_axis_name="s",
                                  num_cores=NC, num_subcores=NS)


@functools.cache
def _deg_kernel():
    @functools.partial(
        pl.kernel,
        out_type=jax.ShapeDtypeStruct((NC, NP, D), jnp.float32),
        mesh=_sc_mesh(),
        scratch_types=[
            pltpu.VMEM((CPW, CH), jnp.int32),
            pltpu.VMEM((CH, D), jnp.float32),
            pltpu.VMEM_SHARED((NP, D), jnp.float32),
            pltpu.SemaphoreType.DMA,
        ],
    )
    def _deg(dst_hbm, ones_hbm, zeros_hbm, out_hbm, dstv, onesv, degsp, dsem):
        c = lax.axis_index("c")
        s = lax.axis_index("s")
        pltpu.sync_copy(ones_hbm, onesv)
        pltpu.sync_copy(zeros_hbm.at[pl.ds(s * RPT, RPT)],
                        degsp.at[pl.ds(s * RPT, RPT)])
        plsc.subcore_barrier()
        base = (c * NS + s) * CPW
        pltpu.sync_copy(dst_hbm.at[pl.ds(base, CPW)], dstv)

        @pl.loop(0, CPW)
        def _(j):
            pltpu.async_copy(onesv, degsp.at[dstv.at[j]], dsem, add=True)

        @pl.loop(0, CPW)
        def _(j):
            pltpu.make_async_copy(onesv, degsp.at[dstv.at[j]], dsem).wait()

        plsc.subcore_barrier()
        pltpu.sync_copy(degsp.at[pl.ds(s * RPT, RPT)],
                        out_hbm.at[c, pl.ds(s * RPT, RPT)])

    return _deg


SPLIT0 = 256
SPLIT1 = 2 * CPW - SPLIT0
CPWMAX = max(SPLIT0, SPLIT1)
NCH_PAD = NCHUNK + 256


@functools.cache
def _make_scatter(dc):
    @functools.partial(
        pl.kernel,
        out_type=jax.ShapeDtypeStruct((NC, NP, dc), jnp.float32),
        mesh=_sc_mesh(),
        scratch_types=[
            pltpu.VMEM((CPWMAX, CH), jnp.int32),
            pltpu.VMEM((CPWMAX, CH), jnp.int32),
            pltpu.VMEM((2, CH, dc), jnp.float32),
            pltpu.SemaphoreType.DMA((2,)),
            pltpu.VMEM_SHARED((NP, dc), jnp.float32),
        ],
        compiler_params=pltpu.CompilerParams(use_tc_tiling_on_sc=False),
    )
    def _scatter(src_hbm, dst_hbm, z_hbm, zeros_hbm, out_hbm,
                 srcv, dstv, rows, gsems, acc):
        c = lax.axis_index("c")
        s = lax.axis_index("s")
        cpw = jnp.where(c == 0, SPLIT0, SPLIT1)
        base = jnp.where(c == 0, s * SPLIT0, NS * SPLIT0 + s * SPLIT1)
        pltpu.sync_copy(src_hbm.at[pl.ds(base, CPWMAX)], srcv)
        pltpu.sync_copy(dst_hbm.at[pl.ds(base, CPWMAX)], dstv)
        pltpu.sync_copy(zeros_hbm.at[pl.ds(s * RPT, RPT)],
                        acc.at[pl.ds(s * RPT, RPT)])
        plsc.subcore_barrier()

        def _gather(g, b):
            pltpu.async_copy(z_hbm.at[srcv.at[g]], rows.at[b], gsems.at[b])

        def _gwait(g, b):
            pltpu.make_async_copy(z_hbm.at[srcv.at[g]], rows.at[b],
                                  gsems.at[b]).wait()

        @pl.loop(0, CPWMAX + 1)
        def _(g):
            @pl.when(g < cpw)
            def _():
                _gather(g, lax.rem(g, 2))

            @pl.when((g > 0) & (g <= cpw))
            def _():
                b = lax.rem(g + 1, 2)
                _gwait(g - 1, b)
                pltpu.sync_copy(rows.at[b], acc.at[dstv.at[g - 1]], add=True)

        plsc.subcore_barrier()
        pltpu.sync_copy(acc.at[pl.ds(s * RPT, RPT)],
                        out_hbm.at[c, pl.ds(s * RPT, RPT)])

    return _scatter


def _tc_first(x_ref, w_ref, degp_ref, z_ref, dinv_ref):
    dp = degp_ref[0] + degp_ref[1]
    deg = dp[:, 0:1] + 1.0
    dinv = lax.rsqrt(jnp.maximum(deg, 1.0))
    dinvb = jnp.broadcast_to(dinv, (BM, D))
    xw = jnp.dot(x_ref[...], w_ref[...], preferred_element_type=jnp.float32)
    z_ref[...] = dinvb * xw
    dinv_ref[...] = dinvb


def _tc_mid(a_ref, z_ref, dinv_ref, b_ref, g_ref, be_ref, w_ref, zo_ref, dc):
    dinvb = dinv_ref[...]
    u = dinvb * (a_ref[0] + a_ref[1] + z_ref[...]) + b_ref[...]
    u = u * (g_ref[...] / jnp.sqrt(1.0 + EPS)) + be_ref[...]
    u = jnp.maximum(u, 0.0)
    zw = jnp.dot(u, w_ref[...], preferred_element_type=jnp.float32)
    zo_ref[...] = dinvb[:, :dc] * zw


def _tc_last(a_ref, z_ref, dinv_ref, b_ref, o_ref):
    logits = dinv_ref[...] * (a_ref[0] + a_ref[1] + z_ref[...]) + b_ref[...]
    col = lax.broadcasted_iota(jnp.int32, (BM, DOP), 1)
    valid = col < DO
    masked = jnp.where(valid, logits, -1e30)
    m = jnp.max(masked, axis=1, keepdims=True)
    sh = logits - m
    ex = jnp.where(valid, jnp.exp(sh), 0.0)
    lse = jnp.log(jnp.sum(ex, axis=1, keepdims=True))
    o_ref[...] = sh - lse


def _row_spec(dc):
    return pl.BlockSpec((BM, dc), lambda i: (i, 0))


def _full_spec(shape):
    return pl.BlockSpec(shape, lambda i: tuple(0 for _ in shape))


def _pair_spec(dc):
    return pl.BlockSpec((2, BM, dc), lambda i: (0, i, 0))


_GRID = (NP // BM,)


def kernel(x, adj_t, W1, b1, W2, b2, W3, b3, g1, be1, g2, be2):
    f32 = jnp.float32
    src = jnp.concatenate(
        [adj_t[0],
         jnp.zeros((NCH_PAD * CH - E,), jnp.int32)]).reshape(NCH_PAD, CH)
    dst = jnp.concatenate(
        [adj_t[1], jnp.full((EP - E,), N, jnp.int32),
         jnp.zeros(((NCH_PAD - NCHUNK) * CH,), jnp.int32)]).reshape(NCH_PAD,
                                                                    CH)
    x_p = jnp.zeros((NP, D), f32).at[:N].set(x)
    w3p = jnp.zeros((D, DOP), f32).at[:, :DO].set(W3)
    b3p = jnp.zeros((DOP,), f32).at[:DO].set(b3)
    ones_src = jnp.ones((CH, D), f32)
    zeros128 = jnp.zeros((NP, D), f32)
    zeros64 = jnp.zeros((NP, DOP), f32)

    scatter128 = _make_scatter(D)
    degp = _deg_kernel()(dst, ones_src, zeros128)

    z1, dinvb = pl.pallas_call(
        _tc_first,
        grid=_GRID,
        in_specs=[_row_spec(D), _full_spec((D, D)), _pair_spec(D)],
        out_specs=[_row_spec(D), _row_spec(D)],
        out_shape=[jax.ShapeDtypeStruct((NP, D), f32),
                   jax.ShapeDtypeStruct((NP, D), f32)],
    )(x_p, W1, degp)

    a1 = scatter128(src, dst, z1, zeros128)

    z2 = pl.pallas_call(
        functools.partial(_tc_mid, dc=D),
        grid=_GRID,
        in_specs=[_pair_spec(D), _row_spec(D), _row_spec(D),
                  _full_spec((D,)), _full_spec((D,)), _full_spec((D,)),
                  _full_spec((D, D))],
        out_specs=_row_spec(D),
        out_shape=jax.ShapeDtypeStruct((NP, D), f32),
    )(a1, z1, dinvb, b1, g1, be1, W2)

    a2 = scatter128(src, dst, z2, zeros128)

    z3 = pl.pallas_call(
        functools.partial(_tc_mid, dc=DOP),
        grid=_GRID,
        in_specs=[_pair_spec(D), _row_spec(D), _row_spec(D),
                  _full_spec((D,)), _full_spec((D,)), _full_spec((D,)),
                  _full_spec((D, DOP))],
        out_specs=_row_spec(DOP),
        out_shape=jax.ShapeDtypeStruct((NP, DOP), f32),
    )(a2, z2, dinvb, b2, g2, be2, w3p)

    a3 = scatter128(src, dst, z3, zeros64)

    out = pl.pallas_call(
        _tc_last,
        grid=_GRID,
        in_specs=[_pair_spec(DOP), _row_spec(DOP), _row_spec(DOP),
                  _full_spec((DOP,))],
        out_specs=_row_spec(DOP),
        out_shape=jax.ShapeDtypeStruct((NP, DOP), f32),
    )(a3, z3, dinvb, b3p)

    return out[:N, :DO]
